# baseline (device time: 191226 ns/iter reference)
import jax
import jax.numpy as jnp
from jax import lax
from jax.experimental import pallas as pl
from jax.experimental.pallas import tpu as pltpu

N_DEV = 4
E_GLOBAL = 16
E_LOC = 4
CAP = 409.0
T = 2048
D = 512
H = 1024
LANES = 128


def _body(x_ref, idx_ref, ew_ref, out_ref, wall_ref, cnt_ref,
          wsend, wrecv, csend, crecv):
    my = lax.axis_index("i")
    left = lax.rem(my + N_DEV - 1, N_DEV)
    right = lax.rem(my + 1, N_DEV)

    barrier = pltpu.get_barrier_semaphore()
    for nbr in (left, right):
        pl.semaphore_signal(barrier, inc=1, device_id=(nbr,),
                            device_id_type=pl.DeviceIdType.MESH)
    pl.semaphore_wait(barrier, 2)

    wall_ref[pl.ds(my * E_LOC, E_LOC), :, :] = ew_ref[:, :, :].astype(jnp.bfloat16)

    e_iota = lax.broadcasted_iota(jnp.int32, (T, LANES), 1)
    eq = (idx_ref[:, 0:1] == e_iota).astype(jnp.float32)
    counts = jnp.sum(eq, axis=0, keepdims=True)
    cnt_ref[pl.ds(my, 1), :, :] = jnp.broadcast_to(
        counts.reshape(1, 1, LANES), (1, 8, LANES))

    row = lax.broadcasted_iota(jnp.int32, (T, LANES), 0)
    cum = eq
    shift = 1
    while shift < T:
        rolled = pltpu.roll(cum, shift, 0)
        cum = cum + jnp.where(row >= shift, rolled, 0.0)
        shift *= 2
    rank = cum - eq

    for h in range(N_DEV - 1):
        src = lax.rem(my - h + N_DEV, N_DEV)
        rcv = lax.rem(my - h - 1 + N_DEV, N_DEV)
        w_out = pltpu.make_async_remote_copy(
            src_ref=wall_ref.at[pl.ds(src * E_LOC, E_LOC)],
            dst_ref=wall_ref.at[pl.ds(src * E_LOC, E_LOC)],
            send_sem=wsend.at[h], recv_sem=wrecv.at[h],
            device_id=(right,), device_id_type=pl.DeviceIdType.MESH)
        c_out = pltpu.make_async_remote_copy(
            src_ref=cnt_ref.at[pl.ds(src, 1)],
            dst_ref=cnt_ref.at[pl.ds(src, 1)],
            send_sem=csend.at[h], recv_sem=crecv.at[h],
            device_id=(right,), device_id_type=pl.DeviceIdType.MESH)
        w_out.start()
        c_out.start()
        w_in = pltpu.make_async_remote_copy(
            src_ref=wall_ref.at[pl.ds(rcv * E_LOC, E_LOC)],
            dst_ref=wall_ref.at[pl.ds(rcv * E_LOC, E_LOC)],
            send_sem=wsend.at[h], recv_sem=wrecv.at[h],
            device_id=(right,), device_id_type=pl.DeviceIdType.MESH)
        c_in = pltpu.make_async_remote_copy(
            src_ref=cnt_ref.at[pl.ds(rcv, 1)],
            dst_ref=cnt_ref.at[pl.ds(rcv, 1)],
            send_sem=csend.at[h], recv_sem=crecv.at[h],
            device_id=(right,), device_id_type=pl.DeviceIdType.MESH)
        w_out.wait_send()
        c_out.wait_send()
        w_in.wait_recv()
        c_in.wait_recv()

    rows4 = lax.broadcasted_iota(jnp.int32, (N_DEV, 1, LANES), 0)
    offs = jnp.sum(jnp.where(rows4 < my, cnt_ref[:, 0:1, :], 0.0),
                   axis=0, keepdims=False)
    ok = (offs + rank) < CAP
    amask = jnp.where(ok, eq, 0.0).astype(jnp.bfloat16)

    xb = x_ref[:, :].astype(jnp.bfloat16)
    acc = jnp.zeros((T, H), jnp.float32)
    for e in range(E_GLOBAL):
        xm = xb * amask[:, e:e + 1]
        acc = acc + lax.dot_general(
            xm, wall_ref[e],
            dimension_numbers=(((1,), (0,)), ((), ())),
            preferred_element_type=jnp.float32)
    out_ref[:, :] = acc


def kernel(x, router_W, route_idx, expert_W):
    del router_W
    return pl.pallas_call(
        _body,
        out_shape=jax.ShapeDtypeStruct((T, H), jnp.float32),
        in_specs=[
            pl.BlockSpec(memory_space=pltpu.VMEM),
            pl.BlockSpec(memory_space=pltpu.VMEM),
            pl.BlockSpec(memory_space=pltpu.VMEM),
        ],
        out_specs=pl.BlockSpec(memory_space=pltpu.VMEM),
        scratch_shapes=[
            pltpu.VMEM((E_GLOBAL, D, H), jnp.bfloat16),
            pltpu.VMEM((N_DEV, 8, LANES), jnp.float32),
            pltpu.SemaphoreType.DMA((N_DEV - 1,)),
            pltpu.SemaphoreType.DMA((N_DEV - 1,)),
            pltpu.SemaphoreType.DMA((N_DEV - 1,)),
            pltpu.SemaphoreType.DMA((N_DEV - 1,)),
        ],
        compiler_params=pltpu.CompilerParams(collective_id=0),
    )(x, route_idx, expert_W)


# device time: 106928 ns/iter; 1.7884x vs baseline; 1.7884x over previous
import jax
import jax.numpy as jnp
from jax import lax
from jax.experimental import pallas as pl
from jax.experimental.pallas import tpu as pltpu

N_DEV = 4
E_GLOBAL = 16
E_LOC = 4
CAP = 409.0
T = 2048
D = 512
H = 1024
LANES = 128


def _body(x_ref, idx_ref, ew_ref, out_ref, wall_ref, cnt_ref,
          wsR, wrR, wsL, wrL, csend, crecv):
    my = lax.axis_index("i")
    left = lax.rem(my + N_DEV - 1, N_DEV)
    right = lax.rem(my + 1, N_DEV)
    diag = lax.rem(my + 2, N_DEV)

    barrier = pltpu.get_barrier_semaphore()
    for nbr in (left, right):
        pl.semaphore_signal(barrier, inc=1, device_id=(nbr,),
                            device_id_type=pl.DeviceIdType.MESH)
    pl.semaphore_wait(barrier, 2)

    def wcopy(lo, n, sem_s, sem_r, dev):
        return pltpu.make_async_remote_copy(
            src_ref=wall_ref.at[pl.ds(lo, n)],
            dst_ref=wall_ref.at[pl.ds(lo, n)],
            send_sem=sem_s, recv_sem=sem_r,
            device_id=(dev,), device_id_type=pl.DeviceIdType.MESH)

    wall_ref[pl.ds(my * E_LOC, E_LOC), :, :] = ew_ref[:, :, :].astype(jnp.bfloat16)
    h1R = wcopy(my * E_LOC, E_LOC, wsR.at[0], wrR.at[0], right)
    h1L = wcopy(my * E_LOC, E_LOC, wsL.at[0], wrL.at[0], left)
    h1R.start()
    h1L.start()

    e_iota = lax.broadcasted_iota(jnp.int32, (T, LANES), 1)
    eq = (idx_ref[:, 0:1] == e_iota).astype(jnp.float32)
    counts = jnp.sum(eq, axis=0, keepdims=True)
    cnt_ref[pl.ds(my, 1), :, :] = jnp.broadcast_to(
        counts.reshape(1, 1, LANES), (1, 8, LANES))

    for h in range(N_DEV - 1):
        src = lax.rem(my - h + N_DEV, N_DEV)
        rcv = lax.rem(my - h - 1 + N_DEV, N_DEV)
        c_out = pltpu.make_async_remote_copy(
            src_ref=cnt_ref.at[pl.ds(src, 1)],
            dst_ref=cnt_ref.at[pl.ds(src, 1)],
            send_sem=csend.at[h], recv_sem=crecv.at[h],
            device_id=(right,), device_id_type=pl.DeviceIdType.MESH)
        c_out.start()
        c_in = pltpu.make_async_remote_copy(
            src_ref=cnt_ref.at[pl.ds(rcv, 1)],
            dst_ref=cnt_ref.at[pl.ds(rcv, 1)],
            send_sem=csend.at[h], recv_sem=crecv.at[h],
            device_id=(right,), device_id_type=pl.DeviceIdType.MESH)
        c_out.wait_send()
        c_in.wait_recv()

    row = lax.broadcasted_iota(jnp.int32, (T, LANES), 0)
    cum = eq
    shift = 1
    while shift < T:
        rolled = pltpu.roll(cum, shift, 0)
        cum = cum + jnp.where(row >= shift, rolled, 0.0)
        shift *= 2
    rank = cum - eq

    rows4 = lax.broadcasted_iota(jnp.int32, (N_DEV, 1, LANES), 0)
    offs = jnp.sum(jnp.where(rows4 < my, cnt_ref[:, 0:1, :], 0.0),
                   axis=0)
    ok = (offs + rank) < CAP
    accept = jnp.sum(jnp.where(ok, eq, 0.0), axis=1, keepdims=True)
    xb = (x_ref[:, :] * accept).astype(jnp.bfloat16)

    idx = idx_ref[:, 0:1]

    def chunk_matmul(o):
        for j in range(E_LOC):
            e = o * E_LOC + j
            m = (idx == e).astype(jnp.bfloat16)
            w = wall_ref[pl.ds(e, 1), :, :].reshape(D, H)
            out_ref[:, :] = out_ref[:, :] + lax.dot_general(
                xb * m, w,
                dimension_numbers=(((1,), (0,)), ((), ())),
                preferred_element_type=jnp.float32)

    out_ref[:, :] = jnp.zeros((T, H), jnp.float32)
    chunk_matmul(my)

    h1R.wait_send()
    h1L.wait_send()
    r1L = wcopy(left * E_LOC, E_LOC, wsR.at[0], wrR.at[0], right)
    r1R = wcopy(right * E_LOC, E_LOC, wsL.at[0], wrL.at[0], left)
    r1L.wait_recv()
    r1R.wait_recv()

    h2R = wcopy(left * E_LOC, 2, wsR.at[1], wrR.at[1], right)
    h2L = wcopy(right * E_LOC + 2, 2, wsL.at[1], wrL.at[1], left)
    h2R.start()
    h2L.start()

    chunk_matmul(left)
    chunk_matmul(right)

    h2R.wait_send()
    h2L.wait_send()
    r2a = wcopy(diag * E_LOC, 2, wsR.at[1], wrR.at[1], right)
    r2b = wcopy(diag * E_LOC + 2, 2, wsL.at[1], wrL.at[1], left)
    r2a.wait_recv()
    r2b.wait_recv()

    chunk_matmul(diag)


def kernel(x, router_W, route_idx, expert_W):
    del router_W
    return pl.pallas_call(
        _body,
        out_shape=jax.ShapeDtypeStruct((T, H), jnp.float32),
        in_specs=[
            pl.BlockSpec(memory_space=pltpu.VMEM),
            pl.BlockSpec(memory_space=pltpu.VMEM),
            pl.BlockSpec(memory_space=pltpu.VMEM),
        ],
        out_specs=pl.BlockSpec(memory_space=pltpu.VMEM),
        scratch_shapes=[
            pltpu.VMEM((E_GLOBAL, D, H), jnp.bfloat16),
            pltpu.VMEM((N_DEV, 8, LANES), jnp.float32),
            pltpu.SemaphoreType.DMA((2,)),
            pltpu.SemaphoreType.DMA((2,)),
            pltpu.SemaphoreType.DMA((2,)),
            pltpu.SemaphoreType.DMA((2,)),
            pltpu.SemaphoreType.DMA((N_DEV - 1,)),
            pltpu.SemaphoreType.DMA((N_DEV - 1,)),
        ],
        compiler_params=pltpu.CompilerParams(collective_id=0),
    )(x, route_idx, expert_W)


# device time: 106774 ns/iter; 1.7909x vs baseline; 1.0014x over previous
import jax
import jax.numpy as jnp
from jax import lax
from jax.experimental import pallas as pl
from jax.experimental.pallas import tpu as pltpu

N_DEV = 4
E_GLOBAL = 16
E_LOC = 4
CAP = 409.0
T = 2048
D = 512
H = 1024
LANES = 128


def _body(x_ref, idx_ref, ew_ref, out_ref, wall_ref, cnt_ref,
          wsR, wrR, wsL, wrL, csend, crecv):
    my = lax.axis_index("i")
    left = lax.rem(my + N_DEV - 1, N_DEV)
    right = lax.rem(my + 1, N_DEV)
    diag = lax.rem(my + 2, N_DEV)

    barrier = pltpu.get_barrier_semaphore()
    for nbr in (left, right):
        pl.semaphore_signal(barrier, inc=1, device_id=(nbr,),
                            device_id_type=pl.DeviceIdType.MESH)
    pl.semaphore_wait(barrier, 2)

    def wcopy(src_lo, dst_lo, n, sem_s, sem_r, dev):
        return pltpu.make_async_remote_copy(
            src_ref=wall_ref.at[pl.ds(src_lo, n)],
            dst_ref=wall_ref.at[pl.ds(dst_lo, n)],
            send_sem=sem_s, recv_sem=sem_r,
            device_id=(dev,), device_id_type=pl.DeviceIdType.MESH)

    wall_ref[0:E_LOC, :, :] = ew_ref[:, :, :].astype(jnp.bfloat16)
    h1R = wcopy(0, 3 * E_LOC, E_LOC, wsR.at[0], wrR.at[0], right)
    h1L = wcopy(0, 1 * E_LOC, E_LOC, wsL.at[0], wrL.at[0], left)
    h1R.start()
    h1L.start()

    e_iota = lax.broadcasted_iota(jnp.int32, (T, LANES), 1)
    eq = (idx_ref[:, 0:1] == e_iota).astype(jnp.float32)
    counts = jnp.sum(eq, axis=0, keepdims=True)
    cnt_ref[pl.ds(my, 1), :, :] = jnp.broadcast_to(
        counts.reshape(1, 1, LANES), (1, 8, LANES))

    for h in range(N_DEV - 1):
        src = lax.rem(my - h + N_DEV, N_DEV)
        rcv = lax.rem(my - h - 1 + N_DEV, N_DEV)
        c_out = pltpu.make_async_remote_copy(
            src_ref=cnt_ref.at[pl.ds(src, 1)],
            dst_ref=cnt_ref.at[pl.ds(src, 1)],
            send_sem=csend.at[h], recv_sem=crecv.at[h],
            device_id=(right,), device_id_type=pl.DeviceIdType.MESH)
        c_out.start()
        c_in = pltpu.make_async_remote_copy(
            src_ref=cnt_ref.at[pl.ds(rcv, 1)],
            dst_ref=cnt_ref.at[pl.ds(rcv, 1)],
            send_sem=csend.at[h], recv_sem=crecv.at[h],
            device_id=(right,), device_id_type=pl.DeviceIdType.MESH)
        c_out.wait_send()
        c_in.wait_recv()

    row = lax.broadcasted_iota(jnp.int32, (T, LANES), 0)
    cum = eq
    shift = 1
    while shift < T:
        rolled = pltpu.roll(cum, shift, 0)
        cum = cum + jnp.where(row >= shift, rolled, 0.0)
        shift *= 2
    rank = cum - eq

    rows4 = lax.broadcasted_iota(jnp.int32, (N_DEV, 1, LANES), 0)
    offs = jnp.sum(jnp.where(rows4 < my, cnt_ref[:, 0:1, :], 0.0),
                   axis=0)
    ok = (offs + rank) < CAP
    accept = jnp.sum(jnp.where(ok, eq, 0.0), axis=1, keepdims=True)
    xb = (x_ref[:, :] * accept).astype(jnp.bfloat16)

    idx = idx_ref[:, 0:1]

    def chunk_matmul(slot, origin, first=False):
        acc = None
        for j in range(E_LOC):
            m = (idx == origin * E_LOC + j).astype(jnp.bfloat16)
            d = lax.dot_general(
                xb * m, wall_ref[slot * E_LOC + j],
                dimension_numbers=(((1,), (0,)), ((), ())),
                preferred_element_type=jnp.float32)
            acc = d if acc is None else acc + d
        out_ref[:, :] = acc if first else out_ref[:, :] + acc

    chunk_matmul(0, my, first=True)

    h1R.wait_send()
    h1L.wait_send()
    r1L = wcopy(3 * E_LOC, 3 * E_LOC, E_LOC, wsR.at[0], wrR.at[0], right)
    r1R = wcopy(1 * E_LOC, 1 * E_LOC, E_LOC, wsL.at[0], wrL.at[0], left)
    r1L.wait_recv()
    r1R.wait_recv()

    h2R = wcopy(3 * E_LOC, 2 * E_LOC, 2, wsR.at[1], wrR.at[1], right)
    h2L = wcopy(1 * E_LOC + 2, 2 * E_LOC + 2, 2, wsL.at[1], wrL.at[1], left)
    h2R.start()
    h2L.start()

    chunk_matmul(3, left)
    chunk_matmul(1, right)

    h2R.wait_send()
    h2L.wait_send()
    r2a = wcopy(2 * E_LOC, 2 * E_LOC, 2, wsR.at[1], wrR.at[1], right)
    r2b = wcopy(2 * E_LOC + 2, 2 * E_LOC + 2, 2, wsL.at[1], wrL.at[1], left)
    r2a.wait_recv()
    r2b.wait_recv()

    chunk_matmul(2, diag)


def kernel(x, router_W, route_idx, expert_W):
    del router_W
    return pl.pallas_call(
        _body,
        out_shape=jax.ShapeDtypeStruct((T, H), jnp.float32),
        in_specs=[
            pl.BlockSpec(memory_space=pltpu.VMEM),
            pl.BlockSpec(memory_space=pltpu.VMEM),
            pl.BlockSpec(memory_space=pltpu.VMEM),
        ],
        out_specs=pl.BlockSpec(memory_space=pltpu.VMEM),
        scratch_shapes=[
            pltpu.VMEM((E_GLOBAL, D, H), jnp.bfloat16),
            pltpu.VMEM((N_DEV, 8, LANES), jnp.float32),
            pltpu.SemaphoreType.DMA((2,)),
            pltpu.SemaphoreType.DMA((2,)),
            pltpu.SemaphoreType.DMA((2,)),
            pltpu.SemaphoreType.DMA((2,)),
            pltpu.SemaphoreType.DMA((N_DEV - 1,)),
            pltpu.SemaphoreType.DMA((N_DEV - 1,)),
        ],
        compiler_params=pltpu.CompilerParams(collective_id=0),
    )(x, route_idx, expert_W)


# device time: 97061 ns/iter; 1.9702x vs baseline; 1.1001x over previous
import jax
import jax.numpy as jnp
from jax import lax
from jax.experimental import pallas as pl
from jax.experimental.pallas import tpu as pltpu

N_DEV = 4
E_GLOBAL = 16
E_LOC = 4
CAP = 409.0
T = 2048
D = 512
H = 1024
LANES = 128


def _body(x_ref, idx_ref, ew_ref, out_ref, wall_ref, cnt_ref,
          wsR, wrR, wsL, wrL, csend, crecv):
    my = lax.axis_index("i")
    left = lax.rem(my + N_DEV - 1, N_DEV)
    right = lax.rem(my + 1, N_DEV)
    diag = lax.rem(my + 2, N_DEV)

    barrier = pltpu.get_barrier_semaphore()
    for nbr in (left, right):
        pl.semaphore_signal(barrier, inc=1, device_id=(nbr,),
                            device_id_type=pl.DeviceIdType.MESH)
    pl.semaphore_wait(barrier, 2)

    e_iota = lax.broadcasted_iota(jnp.int32, (T, LANES), 1)
    eq = (idx_ref[:, 0:1] == e_iota).astype(jnp.float32)
    counts = jnp.sum(eq, axis=0, keepdims=True)
    cnt_ref[pl.ds(my, 1), :, :] = jnp.broadcast_to(
        counts.reshape(1, 1, LANES), (1, 8, LANES))
    for h in range(N_DEV - 1):
        src = lax.rem(my - h + N_DEV, N_DEV)
        rcv = lax.rem(my - h - 1 + N_DEV, N_DEV)
        c_out = pltpu.make_async_remote_copy(
            src_ref=cnt_ref.at[pl.ds(src, 1)],
            dst_ref=cnt_ref.at[pl.ds(src, 1)],
            send_sem=csend.at[h], recv_sem=crecv.at[h],
            device_id=(right,), device_id_type=pl.DeviceIdType.MESH)
        c_out.start()
        c_in = pltpu.make_async_remote_copy(
            src_ref=cnt_ref.at[pl.ds(rcv, 1)],
            dst_ref=cnt_ref.at[pl.ds(rcv, 1)],
            send_sem=csend.at[h], recv_sem=crecv.at[h],
            device_id=(right,), device_id_type=pl.DeviceIdType.MESH)
        c_out.wait_send()
        c_in.wait_recv()

    def wcopy(src_lo, dst_lo, n, sem_s, sem_r, dev):
        return pltpu.make_async_remote_copy(
            src_ref=wall_ref.at[pl.ds(src_lo, n)],
            dst_ref=wall_ref.at[pl.ds(dst_lo, n)],
            send_sem=sem_s, recv_sem=sem_r,
            device_id=(dev,), device_id_type=pl.DeviceIdType.MESH)

    wall_ref[0:E_LOC, :, :] = ew_ref[:, :, :].astype(jnp.bfloat16)
    h1a = wcopy(0, 12, 2, wsR.at[0], wrR.at[0], right)
    h1La = wcopy(2, 6, 2, wsL.at[0], wrL.at[0], left)
    h1a.start()
    h1La.start()
    h1b = wcopy(2, 14, 2, wsR.at[1], wrR.at[1], right)
    h1Lb = wcopy(0, 4, 2, wsL.at[1], wrL.at[1], left)
    h1b.start()
    h1Lb.start()

    row = lax.broadcasted_iota(jnp.int32, (T, LANES), 0)
    cum = eq
    shift = 1
    while shift < T:
        rolled = pltpu.roll(cum, shift, 0)
        cum = cum + jnp.where(row >= shift, rolled, 0.0)
        shift *= 2
    rank = cum - eq
    rows4 = lax.broadcasted_iota(jnp.int32, (N_DEV, 1, LANES), 0)
    offs = jnp.sum(jnp.where(rows4 < my, cnt_ref[:, 0:1, :], 0.0),
                   axis=0)
    ok = (offs + rank) < CAP
    accept = jnp.sum(jnp.where(ok, eq, 0.0), axis=1, keepdims=True)
    xb = (x_ref[:, :] * accept).astype(jnp.bfloat16)

    idx = idx_ref[:, 0:1]

    def pair_matmul(slot, origin, jlo, first=False):
        acc = None
        for j in (jlo, jlo + 1):
            m = (idx == origin * E_LOC + j).astype(jnp.bfloat16)
            d = lax.dot_general(
                xb * m, wall_ref[slot * E_LOC + j],
                dimension_numbers=(((1,), (0,)), ((), ())),
                preferred_element_type=jnp.float32)
            acc = d if acc is None else acc + d
        out_ref[:, :] = acc if first else out_ref[:, :] + acc

    pair_matmul(0, my, 0, first=True)
    pair_matmul(0, my, 2)

    r1a = wcopy(12, 12, 2, wsR.at[0], wrR.at[0], right)
    r1a.wait_recv()
    h2R = wcopy(12, 8, 2, wsR.at[2], wrR.at[2], right)
    h2R.start()
    r1La = wcopy(6, 6, 2, wsL.at[0], wrL.at[0], left)
    r1La.wait_recv()
    h2L = wcopy(6, 10, 2, wsL.at[2], wrL.at[2], left)
    h2L.start()

    pair_matmul(3, left, 0)
    pair_matmul(1, right, 2)

    r1b = wcopy(14, 14, 2, wsR.at[1], wrR.at[1], right)
    r1Lb = wcopy(4, 4, 2, wsL.at[1], wrL.at[1], left)
    r1b.wait_recv()
    r1Lb.wait_recv()
    pair_matmul(3, left, 2)
    pair_matmul(1, right, 0)

    r2a = wcopy(8, 8, 2, wsR.at[2], wrR.at[2], right)
    r2b = wcopy(10, 10, 2, wsL.at[2], wrL.at[2], left)
    r2a.wait_recv()
    r2b.wait_recv()
    pair_matmul(2, diag, 0)
    pair_matmul(2, diag, 2)

    for s in (h1a, h1b, h1La, h1Lb, h2R, h2L):
        s.wait_send()


def kernel(x, router_W, route_idx, expert_W):
    del router_W
    return pl.pallas_call(
        _body,
        out_shape=jax.ShapeDtypeStruct((T, H), jnp.float32),
        in_specs=[
            pl.BlockSpec(memory_space=pltpu.VMEM),
            pl.BlockSpec(memory_space=pltpu.VMEM),
            pl.BlockSpec(memory_space=pltpu.VMEM),
        ],
        out_specs=pl.BlockSpec(memory_space=pltpu.VMEM),
        scratch_shapes=[
            pltpu.VMEM((E_GLOBAL, D, H), jnp.bfloat16),
            pltpu.VMEM((N_DEV, 8, LANES), jnp.float32),
            pltpu.SemaphoreType.DMA((3,)),
            pltpu.SemaphoreType.DMA((3,)),
            pltpu.SemaphoreType.DMA((3,)),
            pltpu.SemaphoreType.DMA((3,)),
            pltpu.SemaphoreType.DMA((N_DEV - 1,)),
            pltpu.SemaphoreType.DMA((N_DEV - 1,)),
        ],
        compiler_params=pltpu.CompilerParams(collective_id=0),
    )(x, route_idx, expert_W)


# device time: 96756 ns/iter; 1.9764x vs baseline; 1.0032x over previous
import jax
import jax.numpy as jnp
from jax import lax
from jax.experimental import pallas as pl
from jax.experimental.pallas import tpu as pltpu

N_DEV = 4
E_GLOBAL = 16
E_LOC = 4
CAP = 409.0
T = 2048
D = 512
H = 1024
LANES = 128


def _body(x_ref, idx_ref, ew_ref, out_ref, wall_ref, cnt_ref,
          wsR, wrR, wsL, wrL, csend, crecv):
    my = lax.axis_index("i")
    left = lax.rem(my + N_DEV - 1, N_DEV)
    right = lax.rem(my + 1, N_DEV)
    diag = lax.rem(my + 2, N_DEV)

    barrier = pltpu.get_barrier_semaphore()
    for nbr in (left, right):
        pl.semaphore_signal(barrier, inc=1, device_id=(nbr,),
                            device_id_type=pl.DeviceIdType.MESH)
    pl.semaphore_wait(barrier, 2)

    e_iota = lax.broadcasted_iota(jnp.int32, (T, LANES), 1)
    eq = (idx_ref[:, 0:1] == e_iota).astype(jnp.float32)
    counts = jnp.sum(eq, axis=0, keepdims=True)
    cnt_ref[pl.ds(my, 1), :, :] = jnp.broadcast_to(
        counts.reshape(1, 1, LANES), (1, 8, LANES))
    c_outs, c_ins = [], []
    for h in range(N_DEV - 1):
        src = lax.rem(my - h + N_DEV, N_DEV)
        rcv = lax.rem(my - h - 1 + N_DEV, N_DEV)
        c_outs.append(pltpu.make_async_remote_copy(
            src_ref=cnt_ref.at[pl.ds(src, 1)],
            dst_ref=cnt_ref.at[pl.ds(src, 1)],
            send_sem=csend.at[h], recv_sem=crecv.at[h],
            device_id=(right,), device_id_type=pl.DeviceIdType.MESH))
        c_ins.append(pltpu.make_async_remote_copy(
            src_ref=cnt_ref.at[pl.ds(rcv, 1)],
            dst_ref=cnt_ref.at[pl.ds(rcv, 1)],
            send_sem=csend.at[h], recv_sem=crecv.at[h],
            device_id=(right,), device_id_type=pl.DeviceIdType.MESH))
    c_outs[0].start()
    wall_ref[0:E_LOC, :, :] = ew_ref[:, :, :].astype(jnp.bfloat16)
    for h in range(N_DEV - 1):
        if h > 0:
            c_outs[h].start()
        c_outs[h].wait_send()
        c_ins[h].wait_recv()

    def wcopy(src_lo, dst_lo, n, sem_s, sem_r, dev):
        return pltpu.make_async_remote_copy(
            src_ref=wall_ref.at[pl.ds(src_lo, n)],
            dst_ref=wall_ref.at[pl.ds(dst_lo, n)],
            send_sem=sem_s, recv_sem=sem_r,
            device_id=(dev,), device_id_type=pl.DeviceIdType.MESH)

    h1a = wcopy(0, 12, 2, wsR.at[0], wrR.at[0], right)
    h1La = wcopy(2, 6, 2, wsL.at[0], wrL.at[0], left)
    h1a.start()
    h1La.start()
    h1b = wcopy(2, 14, 2, wsR.at[1], wrR.at[1], right)
    h1Lb = wcopy(0, 4, 2, wsL.at[1], wrL.at[1], left)
    h1b.start()
    h1Lb.start()

    row = lax.broadcasted_iota(jnp.int32, (T, LANES), 0)
    cum = eq
    shift = 1
    while shift < T:
        rolled = pltpu.roll(cum, shift, 0)
        cum = cum + jnp.where(row >= shift, rolled, 0.0)
        shift *= 2
    rank = cum - eq
    rows4 = lax.broadcasted_iota(jnp.int32, (N_DEV, 1, LANES), 0)
    offs = jnp.sum(jnp.where(rows4 < my, cnt_ref[:, 0:1, :], 0.0),
                   axis=0)
    ok = (offs + rank) < CAP
    accept = jnp.sum(jnp.where(ok, eq, 0.0), axis=1, keepdims=True)
    xb = (x_ref[:, :] * accept).astype(jnp.bfloat16)

    idx = idx_ref[:, 0:1]

    def masked_matmuls(items, first=False):
        acc = None
        for slot, origin, jlo in items:
            for j in (jlo, jlo + 1):
                m = (idx == origin * E_LOC + j).astype(jnp.bfloat16)
                d = lax.dot_general(
                    xb * m, wall_ref[slot * E_LOC + j],
                    dimension_numbers=(((1,), (0,)), ((), ())),
                    preferred_element_type=jnp.float32)
                acc = d if acc is None else acc + d
        out_ref[:, :] = acc if first else out_ref[:, :] + acc

    masked_matmuls([(0, my, 0), (0, my, 2)], first=True)

    r1a = wcopy(12, 12, 2, wsR.at[0], wrR.at[0], right)
    r1a.wait_recv()
    h2R = wcopy(12, 8, 2, wsR.at[2], wrR.at[2], right)
    h2R.start()
    r1La = wcopy(6, 6, 2, wsL.at[0], wrL.at[0], left)
    r1La.wait_recv()
    h2L = wcopy(6, 10, 2, wsL.at[2], wrL.at[2], left)
    h2L.start()

    masked_matmuls([(3, left, 0), (1, right, 2)])

    r1b = wcopy(14, 14, 2, wsR.at[1], wrR.at[1], right)
    r1Lb = wcopy(4, 4, 2, wsL.at[1], wrL.at[1], left)
    r1b.wait_recv()
    r1Lb.wait_recv()
    masked_matmuls([(3, left, 2), (1, right, 0)])

    r2a = wcopy(8, 8, 2, wsR.at[2], wrR.at[2], right)
    r2b = wcopy(10, 10, 2, wsL.at[2], wrL.at[2], left)
    r2a.wait_recv()
    masked_matmuls([(2, diag, 0)])
    r2b.wait_recv()
    masked_matmuls([(2, diag, 2)])

    for s in (h1a, h1b, h1La, h1Lb, h2R, h2L):
        s.wait_send()


def kernel(x, router_W, route_idx, expert_W):
    del router_W
    return pl.pallas_call(
        _body,
        out_shape=jax.ShapeDtypeStruct((T, H), jnp.float32),
        in_specs=[
            pl.BlockSpec(memory_space=pltpu.VMEM),
            pl.BlockSpec(memory_space=pltpu.VMEM),
            pl.BlockSpec(memory_space=pltpu.VMEM),
        ],
        out_specs=pl.BlockSpec(memory_space=pltpu.VMEM),
        scratch_shapes=[
            pltpu.VMEM((E_GLOBAL, D, H), jnp.bfloat16),
            pltpu.VMEM((N_DEV, 8, LANES), jnp.float32),
            pltpu.SemaphoreType.DMA((3,)),
            pltpu.SemaphoreType.DMA((3,)),
            pltpu.SemaphoreType.DMA((3,)),
            pltpu.SemaphoreType.DMA((3,)),
            pltpu.SemaphoreType.DMA((N_DEV - 1,)),
            pltpu.SemaphoreType.DMA((N_DEV - 1,)),
        ],
        compiler_params=pltpu.CompilerParams(collective_id=0),
    )(x, route_idx, expert_W)


# device time: 93333 ns/iter; 2.0489x vs baseline; 1.0367x over previous
import jax
import jax.numpy as jnp
from jax import lax
from jax.experimental import pallas as pl
from jax.experimental.pallas import tpu as pltpu

N_DEV = 4
E_GLOBAL = 16
E_LOC = 4
CAP = 409.0
T = 2048
D = 512
H = 1024
LANES = 128


def _body(x_ref, idx_ref, ew_ref, out_ref, wall_ref, cnt_ref,
          wsR, wrR, wsL, wrL, csend, crecv):
    my = lax.axis_index("i")
    left = lax.rem(my + N_DEV - 1, N_DEV)
    right = lax.rem(my + 1, N_DEV)
    diag = lax.rem(my + 2, N_DEV)

    barrier = pltpu.get_barrier_semaphore()
    for nbr in (left, right):
        pl.semaphore_signal(barrier, inc=1, device_id=(nbr,),
                            device_id_type=pl.DeviceIdType.MESH)
    pl.semaphore_wait(barrier, 2)

    e_iota = lax.broadcasted_iota(jnp.int32, (T, LANES), 1)
    eq = (idx_ref[:, 0:1] == e_iota).astype(jnp.float32)
    counts = jnp.sum(eq, axis=0, keepdims=True)
    cnt_ref[pl.ds(my, 1), :, :] = jnp.broadcast_to(
        counts.reshape(1, 1, LANES), (1, 8, LANES))
    c_outs, c_ins = [], []
    for h in range(N_DEV - 1):
        src = lax.rem(my - h + N_DEV, N_DEV)
        rcv = lax.rem(my - h - 1 + N_DEV, N_DEV)
        c_outs.append(pltpu.make_async_remote_copy(
            src_ref=cnt_ref.at[pl.ds(src, 1)],
            dst_ref=cnt_ref.at[pl.ds(src, 1)],
            send_sem=csend.at[h], recv_sem=crecv.at[h],
            device_id=(right,), device_id_type=pl.DeviceIdType.MESH))
        c_ins.append(pltpu.make_async_remote_copy(
            src_ref=cnt_ref.at[pl.ds(rcv, 1)],
            dst_ref=cnt_ref.at[pl.ds(rcv, 1)],
            send_sem=csend.at[h], recv_sem=crecv.at[h],
            device_id=(right,), device_id_type=pl.DeviceIdType.MESH))
    c_outs[0].start()
    wall_ref[0:E_LOC, :, :] = ew_ref[:, :, :].astype(jnp.bfloat16)

    def wcopy(src_lo, dst_lo, n, sem_s, sem_r, dev):
        return pltpu.make_async_remote_copy(
            src_ref=wall_ref.at[pl.ds(src_lo, n)],
            dst_ref=wall_ref.at[pl.ds(dst_lo, n)],
            send_sem=sem_s, recv_sem=sem_r,
            device_id=(dev,), device_id_type=pl.DeviceIdType.MESH)

    h1a = wcopy(0, 12, 2, wsR.at[0], wrR.at[0], right)
    h1La = wcopy(2, 6, 2, wsL.at[0], wrL.at[0], left)
    h1a.start()
    h1La.start()
    h1b = wcopy(2, 14, 2, wsR.at[1], wrR.at[1], right)
    h1Lb = wcopy(0, 4, 2, wsL.at[1], wrL.at[1], left)
    h1b.start()
    h1Lb.start()

    row = lax.broadcasted_iota(jnp.int32, (T, LANES), 0)
    cum = eq
    shift = 1
    while shift < T:
        rolled = pltpu.roll(cum, shift, 0)
        cum = cum + jnp.where(row >= shift, rolled, 0.0)
        shift *= 2
    rank = cum - eq
    xb = x_ref[:, :].astype(jnp.bfloat16)

    idx = idx_ref[:, 0:1]

    def masked_matmuls(items, first=False, scale=None):
        acc = None
        for slot, origin, jlo in items:
            for j in (jlo, jlo + 1):
                m = (idx == origin * E_LOC + j).astype(jnp.bfloat16)
                d = lax.dot_general(
                    xb * m, wall_ref[slot * E_LOC + j],
                    dimension_numbers=(((1,), (0,)), ((), ())),
                    preferred_element_type=jnp.float32)
                acc = d if acc is None else acc + d
        total = acc if first else out_ref[:, :] + acc
        out_ref[:, :] = total if scale is None else total * scale

    masked_matmuls([(0, my, 0), (0, my, 2)], first=True)

    c_outs[0].wait_send()
    c_ins[0].wait_recv()
    c_outs[1].start()

    r1a = wcopy(12, 12, 2, wsR.at[0], wrR.at[0], right)
    r1a.wait_recv()
    h2R = wcopy(12, 8, 2, wsR.at[2], wrR.at[2], right)
    h2R.start()
    r1La = wcopy(6, 6, 2, wsL.at[0], wrL.at[0], left)
    r1La.wait_recv()
    h2L = wcopy(6, 10, 2, wsL.at[2], wrL.at[2], left)
    h2L.start()

    masked_matmuls([(3, left, 0), (1, right, 2)])

    c_outs[1].wait_send()
    c_ins[1].wait_recv()
    c_outs[2].start()

    r1b = wcopy(14, 14, 2, wsR.at[1], wrR.at[1], right)
    r1Lb = wcopy(4, 4, 2, wsL.at[1], wrL.at[1], left)
    r1b.wait_recv()
    r1Lb.wait_recv()
    masked_matmuls([(3, left, 2), (1, right, 0)])

    c_outs[2].wait_send()
    c_ins[2].wait_recv()

    rows4 = lax.broadcasted_iota(jnp.int32, (N_DEV, 1, LANES), 0)
    offs = jnp.sum(jnp.where(rows4 < my, cnt_ref[:, 0:1, :], 0.0),
                   axis=0)
    ok = (offs + rank) < CAP
    accept = jnp.sum(jnp.where(ok, eq, 0.0), axis=1, keepdims=True)

    r2a = wcopy(8, 8, 2, wsR.at[2], wrR.at[2], right)
    r2b = wcopy(10, 10, 2, wsL.at[2], wrL.at[2], left)
    r2a.wait_recv()
    masked_matmuls([(2, diag, 0)])
    r2b.wait_recv()
    masked_matmuls([(2, diag, 2)], scale=accept)

    for s in (h1a, h1b, h1La, h1Lb, h2R, h2L):
        s.wait_send()


def kernel(x, router_W, route_idx, expert_W):
    del router_W
    return pl.pallas_call(
        _body,
        out_shape=jax.ShapeDtypeStruct((T, H), jnp.float32),
        in_specs=[
            pl.BlockSpec(memory_space=pltpu.VMEM),
            pl.BlockSpec(memory_space=pltpu.VMEM),
            pl.BlockSpec(memory_space=pltpu.VMEM),
        ],
        out_specs=pl.BlockSpec(memory_space=pltpu.VMEM),
        scratch_shapes=[
            pltpu.VMEM((E_GLOBAL, D, H), jnp.bfloat16),
            pltpu.VMEM((N_DEV, 8, LANES), jnp.float32),
            pltpu.SemaphoreType.DMA((3,)),
            pltpu.SemaphoreType.DMA((3,)),
            pltpu.SemaphoreType.DMA((3,)),
            pltpu.SemaphoreType.DMA((3,)),
            pltpu.SemaphoreType.DMA((N_DEV - 1,)),
            pltpu.SemaphoreType.DMA((N_DEV - 1,)),
        ],
        compiler_params=pltpu.CompilerParams(collective_id=0),
    )(x, route_idx, expert_W)
